# Initial kernel scaffold; baseline (speedup 1.0000x reference)
import jax
import jax.numpy as jnp
from jax.experimental import pallas as pl


def kernel(obj_features, obj_logits, obj_bboxes, Ws1, bs1, Ws2, bs2, Wo1, bo1, Wo2, bo2, Wcs, bcs, Wco, bco, Wps1, bps1, Wps2, bps2, Wpo1, bpo1, Wpo2, bpo2):
    raise NotImplementedError("write your pallas kernel here")


# speedup vs baseline: 2.9128x; 2.9128x over previous
import functools

import jax
import jax.numpy as jnp
from jax.experimental import pallas as pl

_INTERPRET = False

N = 2048
HID = 64
NUM_CLASSES = 151
K = 256
IM_W, IM_H = 1024.0, 681.0
ROW_BLK = 256

_DOT = functools.partial(jnp.dot, preferred_element_type=jnp.float32)


def _mlp(x, W1, b1, W2, b2):
    h = jnp.maximum(_DOT(x, W1) + b1, 0.0)
    return _DOT(h, W2) + b2


def _factors_body(feat_ref, logits_ref, bbox_ref,
                  Ws1_ref, bs1_ref, Ws2_ref, bs2_ref,
                  Wo1_ref, bo1_ref, Wo2_ref, bo2_ref,
                  Wcs_ref, bcs_ref, Wco_ref, bco_ref,
                  Wps1_ref, bps1_ref, Wps2_ref, bps2_ref,
                  Wpo1_ref, bpo1_ref, Wpo2_ref, bpo2_ref,
                  fs_ref, fo_ref):
    feat = feat_ref[...]
    xs = _mlp(feat, Ws1_ref[...], bs1_ref[...], Ws2_ref[...], bs2_ref[...])
    xo = _mlp(feat, Wo1_ref[...], bo1_ref[...], Wo2_ref[...], bo2_ref[...])

    logits = logits_ref[...]
    m = jnp.max(logits, axis=-1, keepdims=True)
    e = jnp.exp(logits - m)
    probs = e / jnp.sum(e, axis=-1, keepdims=True)
    cs = _DOT(probs, Wcs_ref[...]) + bcs_ref[...]
    co = _DOT(probs, Wco_ref[...]) + bco_ref[...]

    bbox = bbox_ref[...]
    x1, y1, x2, y2 = bbox[:, 0:1], bbox[:, 1:2], bbox[:, 2:3], bbox[:, 3:4]
    w = x2 - x1
    h = y2 - y1
    pos = jnp.concatenate([x1 / IM_W, y1 / IM_H, x2 / IM_W, y2 / IM_H,
                           (w * h) / (IM_W * IM_H), w / (h + 1e-6)], axis=1)
    ps = _mlp(pos, Wps1_ref[...], bps1_ref[...], Wps2_ref[...], bps2_ref[...])
    po = _mlp(pos, Wpo1_ref[...], bpo1_ref[...], Wpo2_ref[...], bpo2_ref[...])

    fs_ref[...] = jnp.concatenate([xs, cs, ps], axis=1)
    fo_ref[...] = jnp.concatenate([xo, co, po], axis=1)


def _scores_body(fs_ref, fo_ref, out_ref):
    fs = fs_ref[...]
    fo = fo_ref[...]
    dot_t = functools.partial(jax.lax.dot_general,
                              dimension_numbers=(((1,), (1,)), ((), ())),
                              preferred_element_type=jnp.float32)
    s = dot_t(fs[:, 0:64], fo[:, 0:64])
    s = s + dot_t(fs[:, 64:128], fo[:, 64:128])
    s = s + dot_t(fs[:, 128:192], fo[:, 128:192])
    out_ref[...] = jax.nn.sigmoid(s)


def _compute_relness(obj_features, obj_logits, obj_bboxes, weights):
    nb = N // ROW_BLK
    row_spec = lambda c: pl.BlockSpec((ROW_BLK, c), lambda i: (i, 0))
    full = lambda a: pl.BlockSpec(a.shape, lambda i: (0,) * a.ndim)
    w_specs = [full(w) for w in weights]
    fs, fo = pl.pallas_call(
        _factors_body,
        grid=(nb,),
        in_specs=[row_spec(2048), row_spec(NUM_CLASSES), row_spec(4)] + w_specs,
        out_specs=[pl.BlockSpec((ROW_BLK, 192), lambda i: (i, 0))] * 2,
        out_shape=[jax.ShapeDtypeStruct((N, 192), jnp.float32)] * 2,
        interpret=_INTERPRET,
    )(obj_features, obj_logits, obj_bboxes, *weights)

    relness = pl.pallas_call(
        _scores_body,
        grid=(nb,),
        in_specs=[pl.BlockSpec((ROW_BLK, 192), lambda i: (i, 0)),
                  pl.BlockSpec((N, 192), lambda i: (0, 0))],
        out_specs=pl.BlockSpec((ROW_BLK, N), lambda i: (i, 0)),
        out_shape=jax.ShapeDtypeStruct((N, N), jnp.float32),
        interpret=_INTERPRET,
    )(fs, fo)
    return relness


def kernel(obj_features, obj_logits, obj_bboxes,
           Ws1, bs1, Ws2, bs2, Wo1, bo1, Wo2, bo2,
           Wcs, bcs, Wco, bco,
           Wps1, bps1, Wps2, bps2, Wpo1, bpo1, Wpo2, bpo2):
    weights = (Ws1, bs1, Ws2, bs2, Wo1, bo1, Wo2, bo2, Wcs, bcs, Wco, bco,
               Wps1, bps1, Wps2, bps2, Wpo1, bpo1, Wpo2, bpo2)
    relness = _compute_relness(obj_features, obj_logits, obj_bboxes, weights)

    flat = relness.reshape(-1)
    rel_flat = flat[1:].reshape(N - 1, N + 1)[:, :N].reshape(-1)

    vals, idx = jax.lax.top_k(rel_flat, K)
    i = idx // (N - 1)
    m = idx % (N - 1)
    j = m + (m >= i).astype(idx.dtype)
    boxes = jnp.concatenate([obj_bboxes[i], obj_bboxes[j]], axis=1)
    return rel_flat, vals, boxes


# speedup vs baseline: 27.4940x; 9.4389x over previous
import functools

import jax
import jax.numpy as jnp
from jax import lax
from jax.experimental import pallas as pl
from jax.experimental.pallas import tpu as pltpu
from jax.experimental.pallas import tpu_sc as plsc

_INTERPRET = False

N = 2048
HID = 64
NUM_CLASSES = 151
K = 256
IM_W, IM_H = 1024.0, 681.0
ROW_BLK = 256

NSQ = N * N
HALF = NSQ // 2
NTILE = 16
PER_TILE = HALF // NTILE
CHUNK = 16384
NCHUNK = PER_TILE // CHUNK
VPC = CHUNK // 16

_DOT = functools.partial(jnp.dot, preferred_element_type=jnp.float32)


def _mlp(x, W1, b1, W2, b2):
    h = jnp.maximum(_DOT(x, W1) + b1, 0.0)
    return _DOT(h, W2) + b2


def _factors_body(feat_ref, logits_ref, bbox_ref,
                  Ws1_ref, bs1_ref, Ws2_ref, bs2_ref,
                  Wo1_ref, bo1_ref, Wo2_ref, bo2_ref,
                  Wcs_ref, bcs_ref, Wco_ref, bco_ref,
                  Wps1_ref, bps1_ref, Wps2_ref, bps2_ref,
                  Wpo1_ref, bpo1_ref, Wpo2_ref, bpo2_ref,
                  fs_ref, fo_ref):
    feat = feat_ref[...]
    xs = _mlp(feat, Ws1_ref[...], bs1_ref[...], Ws2_ref[...], bs2_ref[...])
    xo = _mlp(feat, Wo1_ref[...], bo1_ref[...], Wo2_ref[...], bo2_ref[...])

    logits = logits_ref[...]
    m = jnp.max(logits, axis=-1, keepdims=True)
    e = jnp.exp(logits - m)
    probs = e / jnp.sum(e, axis=-1, keepdims=True)
    cs = _DOT(probs, Wcs_ref[...]) + bcs_ref[...]
    co = _DOT(probs, Wco_ref[...]) + bco_ref[...]

    bbox = bbox_ref[...]
    x1, y1, x2, y2 = bbox[:, 0:1], bbox[:, 1:2], bbox[:, 2:3], bbox[:, 3:4]
    w = x2 - x1
    h = y2 - y1
    pos = jnp.concatenate([x1 / IM_W, y1 / IM_H, x2 / IM_W, y2 / IM_H,
                           (w * h) / (IM_W * IM_H), w / (h + 1e-6)], axis=1)
    ps = _mlp(pos, Wps1_ref[...], bps1_ref[...], Wps2_ref[...], bps2_ref[...])
    po = _mlp(pos, Wpo1_ref[...], bpo1_ref[...], Wpo2_ref[...], bpo2_ref[...])

    fs_ref[...] = jnp.concatenate([xs, cs, ps], axis=1)
    fo_ref[...] = jnp.concatenate([xo, co, po], axis=1)


def _scores_body(fs_ref, fo_ref, out_ref):
    fs = fs_ref[...]
    fo = fo_ref[...]
    dot_t = functools.partial(lax.dot_general,
                              dimension_numbers=(((1,), (1,)), ((), ())),
                              preferred_element_type=jnp.float32)
    s = dot_t(fs[:, 0:64], fo[:, 0:64])
    s = s + dot_t(fs[:, 64:128], fo[:, 64:128])
    s = s + dot_t(fs[:, 128:192], fo[:, 128:192])
    rel = jax.nn.sigmoid(s)
    pid = pl.program_id(0)
    row = lax.broadcasted_iota(jnp.int32, (ROW_BLK, N), 0) + pid * ROW_BLK
    col = lax.broadcasted_iota(jnp.int32, (ROW_BLK, N), 1)
    out_ref[...] = jnp.where(row == col, -1.0, rel)


def _compute_relness(obj_features, obj_logits, obj_bboxes, weights):
    nb = N // ROW_BLK
    row_spec = lambda c: pl.BlockSpec((ROW_BLK, c), lambda i: (i, 0))
    full = lambda a: pl.BlockSpec(a.shape, lambda i: (0,) * a.ndim)
    w_specs = [full(w) for w in weights]
    fs, fo = pl.pallas_call(
        _factors_body,
        grid=(nb,),
        in_specs=[row_spec(2048), row_spec(NUM_CLASSES), row_spec(4)] + w_specs,
        out_specs=[pl.BlockSpec((ROW_BLK, 192), lambda i: (i, 0))] * 2,
        out_shape=[jax.ShapeDtypeStruct((N, 192), jnp.float32)] * 2,
        interpret=_INTERPRET,
    )(obj_features, obj_logits, obj_bboxes, *weights)

    relness = pl.pallas_call(
        _scores_body,
        grid=(nb,),
        in_specs=[pl.BlockSpec((ROW_BLK, 192), lambda i: (i, 0)),
                  pl.BlockSpec((N, 192), lambda i: (0, 0))],
        out_specs=pl.BlockSpec((ROW_BLK, N), lambda i: (i, 0)),
        out_shape=jax.ShapeDtypeStruct((N, N), jnp.float32),
        interpret=_INTERPRET,
    )(fs, fo)
    return relness


def _sc_topk_body(a_hbm,
                  tstar_out, gtidx_out, gtval_out, cntgt_out,
                  eqidx_out, cnteq_out,
                  buf0, buf1, hist, totals_v, merge_v,
                  res_b, res_nhi,
                  gtib, gtvb, eqib, splat_v,
                  sh_hist, sh_resb, sh_resnhi,
                  sem0, sem1):
    c = lax.axis_index("c")
    s = lax.axis_index("s")
    tile_start = c * HALF + s * PER_TILE
    lanes = lax.iota(jnp.int32, 16)
    ones_i = jnp.ones((16,), jnp.int32)

    def zero_hist():
        def zbody(i, _):
            hist[pl.ds(i * 16, 16)] = jnp.zeros((16,), jnp.int32)
            return 0
        lax.fori_loop(0, 4096, zbody, 0)

    def sweep_hist(bin_fn, cond_fn):
        def process(buf):
            def vbody(k, _):
                b = buf[pl.ds(k * 16, 16)]
                idx = lanes * 4096 + bin_fn(b)
                plsc.addupdate_scatter(hist, [idx], ones_i, mask=cond_fn(b))
                return 0
            lax.fori_loop(0, VPC, vbody, 0)

        def start(buf, sem, ci):
            pltpu.make_async_copy(
                a_hbm.at[pl.ds(tile_start + ci * CHUNK, CHUNK)], buf, sem
            ).start()

        def wait(buf, sem, ci):
            pltpu.make_async_copy(
                a_hbm.at[pl.ds(tile_start + ci * CHUNK, CHUNK)], buf, sem
            ).wait()

        start(buf0, sem0, 0)

        def cbody(i, _):
            ci = i * 2
            wait(buf0, sem0, ci)
            start(buf1, sem1, ci + 1)
            process(buf0)
            wait(buf1, sem1, ci + 1)

            @pl.when(i < NCHUNK // 2 - 1)
            def _():
                start(buf0, sem0, ci + 2)

            process(buf1)
            return 0
        lax.fori_loop(0, NCHUNK // 2, cbody, 0)

    def merge_and_search(n_hi_prev, nbins):
        def tbody(j, _):
            acc = jnp.zeros((16,), jnp.int32)
            for l in range(16):
                acc = acc + hist[pl.ds(l * 4096 + j * 16, 16)]
            totals_v[pl.ds(j * 16, 16)] = acc
            return 0
        lax.fori_loop(0, nbins // 16, tbody, 0)
        pltpu.sync_copy(totals_v.at[pl.ds(0, nbins)], sh_hist.at[s, pl.ds(0, nbins)])
        plsc.subcore_barrier()

        @pl.when(s == 0)
        def _():
            nseg = max(nbins // 256, 1)
            seg = min(nbins, 256)

            def sbody(i, carry):
                running, b_found, n_hi_new = carry
                cidx = nseg - 1 - i
                pltpu.sync_copy(sh_hist.at[:, pl.ds(cidx * seg, seg)],
                                merge_v.at[pl.ds(0, 16), pl.ds(0, seg)])

                def rbody(r, carry2):
                    running2, b_found2, n_hi2 = carry2
                    rr = seg // 16 - 1 - r
                    bins = jnp.zeros((16,), jnp.int32)
                    for l in range(16):
                        bins = bins + merge_v[l, pl.ds(rr * 16, 16)]
                    total = jnp.sum(bins)
                    pre = plsc.cumsum(bins)
                    suffix = running2 + (total - pre) + bins
                    cond = (n_hi_prev + suffix) >= K
                    lane = jnp.max(jnp.where(cond, lanes, -1))
                    hit = (lane >= 0) & (b_found2 < 0)
                    bsel = cidx * seg + rr * 16 + lane
                    nh = jnp.sum(jnp.where(lanes == lane, suffix - bins, 0))
                    b_new = jnp.where(hit, bsel, b_found2)
                    nh_new = jnp.where(hit, n_hi_prev + nh, n_hi2)
                    return (running2 + total, b_new, nh_new)

                return lax.fori_loop(0, seg // 16, rbody,
                                     (running, b_found, n_hi_new))

            _, b_star, n_hi_new = lax.fori_loop(
                0, nseg, sbody, (jnp.int32(0), jnp.int32(-1), jnp.int32(0)))
            res_b[...] = jnp.full((16,), b_star, jnp.int32)
            res_nhi[...] = jnp.full((16,), n_hi_new, jnp.int32)
            pltpu.sync_copy(res_b, sh_resb)
            pltpu.sync_copy(res_nhi, sh_resnhi)

        plsc.subcore_barrier()
        pltpu.sync_copy(sh_resb, res_b)
        pltpu.sync_copy(sh_resnhi, res_nhi)
        return jnp.max(res_b[...]), jnp.max(res_nhi[...])

    zero_hist()
    sweep_hist(lambda b: lax.shift_right_arithmetic(b, 18),
               lambda b: b >= 0)
    b1, nh1 = merge_and_search(jnp.int32(0), 4096)

    zero_hist()
    b1v = jnp.full((16,), b1, jnp.int32)
    sweep_hist(lambda b: lax.shift_right_arithmetic(b, 6) & 4095,
               lambda b: (b >= 0) & (lax.shift_right_arithmetic(b, 18) == b1v))
    b2, nh2 = merge_and_search(nh1, 4096)

    pre2 = (b1 * 4096 + b2)
    pre2v = jnp.full((16,), pre2, jnp.int32)
    zero_hist()
    sweep_hist(lambda b: b & 63,
               lambda b: (b >= 0) & (lax.shift_right_arithmetic(b, 6) == pre2v))
    b3, _ = merge_and_search(nh2, 256)

    tstar = pre2 * 64 + b3
    tstar_v = jnp.full((16,), tstar, jnp.int32)

    def collect():
        def process(buf, ci, carry):
            def vbody(k, carry2):
                cg, ce = carry2
                b = buf[pl.ds(k * 16, 16)]
                q = tile_start + ci * CHUNK + k * 16 + lanes
                gt_m = b > tstar_v
                eq_m = (b == tstar_v) & (ce < K)
                cg_s = jnp.max(cg)
                ce_s = jnp.max(ce)
                plsc.store_scatter(gtib, [cg_s + plsc.cumsum(gt_m.astype(jnp.int32)) - 1],
                                   q, mask=gt_m)
                plsc.store_scatter(gtvb, [cg_s + plsc.cumsum(gt_m.astype(jnp.int32)) - 1],
                                   b, mask=gt_m)
                plsc.store_scatter(eqib, [ce_s + plsc.cumsum(eq_m.astype(jnp.int32)) - 1],
                                   q, mask=eq_m)
                cg2 = cg + plsc.all_reduce_population_count(gt_m)
                ce2 = ce + plsc.all_reduce_population_count(eq_m)
                return (cg2, ce2)
            return lax.fori_loop(0, VPC, vbody, carry)

        def start(buf, sem, ci):
            pltpu.make_async_copy(
                a_hbm.at[pl.ds(tile_start + ci * CHUNK, CHUNK)], buf, sem
            ).start()

        def wait(buf, sem, ci):
            pltpu.make_async_copy(
                a_hbm.at[pl.ds(tile_start + ci * CHUNK, CHUNK)], buf, sem
            ).wait()

        start(buf0, sem0, 0)

        def cbody(i, carry):
            ci = i * 2
            wait(buf0, sem0, ci)
            start(buf1, sem1, ci + 1)
            carry = process(buf0, ci, carry)
            wait(buf1, sem1, ci + 1)

            @pl.when(i < NCHUNK // 2 - 1)
            def _():
                start(buf0, sem0, ci + 2)

            carry = process(buf1, ci + 1, carry)
            return carry
        z = jnp.zeros((16,), jnp.int32)
        return lax.fori_loop(0, NCHUNK // 2, cbody, (z, z))

    cg, ce = collect()

    pltpu.sync_copy(gtib.at[pl.ds(0, K)], gtidx_out.at[c, s])
    pltpu.sync_copy(gtvb.at[pl.ds(0, K)], gtval_out.at[c, s])
    pltpu.sync_copy(eqib.at[pl.ds(0, K)], eqidx_out.at[c, s])
    splat_v[...] = cg
    pltpu.sync_copy(splat_v, cntgt_out.at[c, s])
    splat_v[...] = jnp.minimum(ce, K)
    pltpu.sync_copy(splat_v, cnteq_out.at[c, s])

    @pl.when(s == 0)
    def _():
        res_b[...] = tstar_v
        pltpu.sync_copy(res_b, tstar_out.at[c])


def _sc_topk(a_flat):
    mesh = plsc.VectorSubcoreMesh(core_axis_name="c", subcore_axis_name="s")
    f = pl.kernel(
        _sc_topk_body,
        mesh=mesh,
        compiler_params=pltpu.CompilerParams(needs_layout_passes=False),
        out_type=[
            jax.ShapeDtypeStruct((2, 16), jnp.int32),
            jax.ShapeDtypeStruct((2, NTILE, K), jnp.int32),
            jax.ShapeDtypeStruct((2, NTILE, K), jnp.int32),
            jax.ShapeDtypeStruct((2, NTILE, 16), jnp.int32),
            jax.ShapeDtypeStruct((2, NTILE, K), jnp.int32),
            jax.ShapeDtypeStruct((2, NTILE, 16), jnp.int32),
        ],
        scratch_types=[
            pltpu.VMEM((CHUNK,), jnp.int32),
            pltpu.VMEM((CHUNK,), jnp.int32),
            pltpu.VMEM((65536,), jnp.int32),
            pltpu.VMEM((4096,), jnp.int32),
            pltpu.VMEM((16, 256), jnp.int32),
            pltpu.VMEM((16,), jnp.int32),
            pltpu.VMEM((16,), jnp.int32),
            pltpu.VMEM((K + 16,), jnp.int32),
            pltpu.VMEM((K + 16,), jnp.int32),
            pltpu.VMEM((K + 16,), jnp.int32),
            pltpu.VMEM((16,), jnp.int32),
            pltpu.VMEM_SHARED((16, 4096), jnp.int32),
            pltpu.VMEM_SHARED((16,), jnp.int32),
            pltpu.VMEM_SHARED((16,), jnp.int32),
            pltpu.SemaphoreType.DMA,
            pltpu.SemaphoreType.DMA,
        ],
    )
    return f(a_flat)


def _assemble_body(gtidx_ref, gtval_ref, cntgt_ref, eqidx_ref, cnteq_ref,
                   tstar_ref, bbox_ref, vals_ref, boxes_ref):
    iota_k = lax.broadcasted_iota(jnp.int32, (K, K), 0)
    iota_kc = lax.broadcasted_iota(jnp.int32, (K, K), 1)
    iota_j = lax.broadcasted_iota(jnp.int32, (NTILE, K), 1)

    cg_all = cntgt_ref[...][:, 0:1]
    ce_all = cnteq_ref[...][:, 0:1]

    half_bits = []
    half_idx = []
    half_val = []
    for h in range(2):
        r0 = h * NTILE
        cg = cg_all[r0:r0 + NTILE]
        ce = ce_all[r0:r0 + NTILE]
        gti = gtidx_ref[...][r0:r0 + NTILE]
        gtb_all = gtval_ref[...][r0:r0 + NTILE]
        eqi = eqidx_ref[...][r0:r0 + NTILE]

        tmask = (lax.broadcasted_iota(jnp.int32, (NTILE, NTILE), 1)
                 < lax.broadcasted_iota(jnp.int32, (NTILE, NTILE), 0))
        pref_g = jnp.sum(jnp.where(tmask, cg.T, 0), axis=1, keepdims=True)
        pref_e = jnp.sum(jnp.where(tmask, ce.T, 0), axis=1, keepdims=True)
        n_gt = jnp.sum(cg)

        pos_g = pref_g + iota_j
        valid_g = iota_j < cg
        pos_e = n_gt + pref_e + iota_j
        valid_e = (iota_j < ce) & (pos_e < K)

        acc_b = jnp.zeros((K,), jnp.int32)
        acc_i = jnp.zeros((K,), jnp.int32)
        eq_i = jnp.zeros((K,), jnp.int32)
        kcol = lax.broadcasted_iota(jnp.int32, (K, K), 0)
        for t in range(NTILE):
            m = (pos_g[t][None, :] == kcol) & valid_g[t][None, :]
            acc_b = acc_b + jnp.sum(jnp.where(m, gtb_all[t][None, :], 0), axis=1)
            acc_i = acc_i + jnp.sum(jnp.where(m, gti[t][None, :], 0), axis=1)
            me = (pos_e[t][None, :] == kcol) & valid_e[t][None, :]
            eq_i = eq_i + jnp.sum(jnp.where(me, eqi[t][None, :], 0), axis=1)

        slot = lax.iota(jnp.int32, K)
        vb = jnp.where(slot < n_gt, acc_b, -1)
        vi = jnp.where(slot < n_gt, acc_i, jnp.int32(1 << 30))

        bj = vb[:, None]
        bi = vb[None, :]
        ij = vi[:, None]
        ii = vi[None, :]
        beats = (bj > bi) | ((bj == bi) & (ij < ii))
        rank = jnp.sum(beats.astype(jnp.int32), axis=0)

        oh = (rank[None, :] == iota_k[:, 0:1])
        srt_b = jnp.sum(jnp.where(oh, vb[None, :], 0), axis=1)
        srt_i = jnp.sum(jnp.where(oh, vi[None, :], 0), axis=1)

        tsb = tstar_ref[h, 0]
        hb = jnp.where(slot < n_gt, srt_b, tsb)
        hi = jnp.where(slot < n_gt, srt_i, eq_i)
        half_bits.append(hb)
        half_idx.append(hi)

    mb = jnp.concatenate(half_bits)
    mi = jnp.concatenate(half_idx)
    bj = mb[:, None]
    bi = mb[None, :]
    ij = mi[:, None]
    ii = mi[None, :]
    beats = (bj > bi) | ((bj == bi) & (ij < ii))
    rank = jnp.sum(beats.astype(jnp.int32), axis=0)

    iota_k512 = lax.broadcasted_iota(jnp.int32, (K, 2 * K), 0)
    oh = (rank[None, :] == iota_k512)
    out_b = jnp.sum(jnp.where(oh, mb[None, :], 0), axis=1)
    out_i = jnp.sum(jnp.where(oh, mi[None, :], 0), axis=1)

    vals_ref[...] = lax.bitcast_convert_type(out_b, jnp.float32)

    bbox = bbox_ref[...]
    si = out_i // N
    oj = out_i - si * N
    iota_n = lax.broadcasted_iota(jnp.int32, (K, N), 1)
    ms = (iota_n == si[:, None])
    mo = (iota_n == oj[:, None])
    cols = []
    for cidx in range(4):
        col = bbox[:, cidx][None, :]
        cols.append(jnp.sum(jnp.where(ms, col, 0.0), axis=1, keepdims=True))
    for cidx in range(4):
        col = bbox[:, cidx][None, :]
        cols.append(jnp.sum(jnp.where(mo, col, 0.0), axis=1, keepdims=True))
    boxes_ref[...] = jnp.concatenate(cols, axis=1)


def _assemble(gtidx, gtval, cntgt, eqidx, cnteq, tstar, bbox):
    full = lambda shape: pl.BlockSpec(shape, lambda: (0,) * len(shape))
    return pl.pallas_call(
        _assemble_body,
        in_specs=[full((32, K)), full((32, K)), full((32, 16)),
                  full((32, K)), full((32, 16)), full((2, 16)),
                  full((N, 4))],
        out_specs=[full((K,)), full((K, 8))],
        out_shape=[jax.ShapeDtypeStruct((K,), jnp.float32),
                   jax.ShapeDtypeStruct((K, 8), jnp.float32)],
        interpret=_INTERPRET,
    )(gtidx, gtval, cntgt, eqidx, cnteq, tstar, bbox)


def kernel(obj_features, obj_logits, obj_bboxes,
           Ws1, bs1, Ws2, bs2, Wo1, bo1, Wo2, bo2,
           Wcs, bcs, Wco, bco,
           Wps1, bps1, Wps2, bps2, Wpo1, bpo1, Wpo2, bpo2):
    weights = (Ws1, bs1, Ws2, bs2, Wo1, bo1, Wo2, bo2, Wcs, bcs, Wco, bco,
               Wps1, bps1, Wps2, bps2, Wpo1, bpo1, Wpo2, bpo2)
    relness = _compute_relness(obj_features, obj_logits, obj_bboxes, weights)

    flat = relness.reshape(-1)
    rel_flat = flat[1:].reshape(N - 1, N + 1)[:, :N].reshape(-1)

    bits = lax.bitcast_convert_type(flat, jnp.int32)
    tstar, gtidx, gtval, cntgt, eqidx, cnteq = _sc_topk(bits)
    vals, boxes = _assemble(gtidx.reshape(32, K), gtval.reshape(32, K),
                            cntgt.reshape(32, 16), eqidx.reshape(32, K),
                            cnteq.reshape(32, 16), tstar, obj_bboxes)
    return rel_flat, vals, boxes
